# baseline (speedup 1.0000x reference)
import functools

import jax
import jax.numpy as jnp
from jax import lax
from jax.experimental import pallas as pl
from jax.experimental.pallas import tpu as pltpu
from jax.experimental.pallas import tpu_sc as plsc

B = 4096
L = 50
LP = 56
DIM = 128
NC = 2
NS = 16
NW = NC * NS
SENT_W = B // NW
CHUNK_S = 2
CHUNK = CHUNK_S * LP
NCHUNK = SENT_W // CHUNK_S
PER_W = SENT_W * LP
NBUF = 4
NGRP = NCHUNK // NBUF


def _emb_body(ids_hbm, table_hbm, out_hbm, idx_v, rows_v, *sems):
    gsems = sems[:NBUF]
    ssems = sems[NBUF:]
    wid = lax.axis_index("s") * NC + lax.axis_index("c")
    base = wid * PER_W
    pltpu.sync_copy(ids_hbm.at[wid], idx_v)

    def gstart(j, b):
        pltpu.make_async_copy(
            table_hbm.at[idx_v.at[j]], rows_v.at[b], gsems[b]).start()

    def gwait(b):
        pltpu.make_async_copy(
            table_hbm.at[idx_v.at[0]], rows_v.at[b], gsems[b]).wait()

    def sstart(j, b):
        pltpu.make_async_copy(
            rows_v.at[b], out_hbm.at[pl.ds(base + j * CHUNK, CHUNK)],
            ssems[b]).start()

    def swait(b):
        pltpu.make_async_copy(
            rows_v.at[b], out_hbm.at[pl.ds(base, CHUNK)], ssems[b]).wait()

    for b in range(NBUF):
        gstart(b, b)

    def body(g, carry):
        j0 = g * NBUF
        for b in range(NBUF):
            gwait(b)
            sstart(j0 + b, b)
        for b in range(NBUF):
            swait(b)
            gstart(j0 + NBUF + b, b)
        return carry

    lax.fori_loop(0, NGRP - 1, body, 0)

    j0 = (NGRP - 1) * NBUF
    for b in range(NBUF):
        gwait(b)
        sstart(j0 + b, b)
    for b in range(NBUF):
        swait(b)


def kernel(word_ids, table):
    ids_p = jnp.pad(word_ids, ((0, 0), (0, LP - L)))
    ids_p = ids_p.reshape(NW, NCHUNK, CHUNK)
    mesh = plsc.VectorSubcoreMesh(core_axis_name="c", subcore_axis_name="s")
    emb = functools.partial(
        pl.kernel,
        mesh=mesh,
        out_type=jax.ShapeDtypeStruct((B * LP, DIM), jnp.float32),
        scratch_types=[
            pltpu.VMEM((NCHUNK, CHUNK), jnp.int32),
            pltpu.VMEM((NBUF, CHUNK, DIM), jnp.float32),
        ] + [pltpu.SemaphoreType.DMA] * (2 * NBUF),
    )(_emb_body)
    out = emb(ids_p, table)
    return out.reshape(B, LP, DIM)[:, :L, :]


# speedup vs baseline: 7.7177x; 7.7177x over previous
import functools

import jax
import jax.numpy as jnp
from jax import lax
from jax.experimental import pallas as pl
from jax.experimental.pallas import tpu as pltpu
from jax.experimental.pallas import tpu_sc as plsc

B = 4096
L = 50
DIM = 128
NC = 2
NS = 16
NW = NC * NS
SENT_W = B // NW
CS = 2
NCHUNK = SENT_W // CS
NBUF = 4
NGRP = NCHUNK // NBUF


def _emb_body(ids_hbm, table_hbm, out_hbm, idx_v, rows_v, *sems):
    gsems = sems[:NBUF]
    ssems = sems[NBUF:]
    wid = lax.axis_index("s") * NC + lax.axis_index("c")
    base = wid * SENT_W
    pltpu.sync_copy(ids_hbm.at[wid], idx_v)

    def gstart(j, b):
        for s in range(CS):
            pltpu.make_async_copy(
                table_hbm.at[idx_v.at[j * CS + s]], rows_v.at[b, s],
                gsems[b]).start()

    def gwait(b):
        for s in range(CS):
            pltpu.make_async_copy(
                table_hbm.at[idx_v.at[0]], rows_v.at[b, s], gsems[b]).wait()

    def sstart(j, b):
        pltpu.make_async_copy(
            rows_v.at[b], out_hbm.at[pl.ds(base + j * CS, CS)],
            ssems[b]).start()

    def swait(b):
        pltpu.make_async_copy(
            rows_v.at[b], out_hbm.at[pl.ds(base, CS)], ssems[b]).wait()

    for b in range(NBUF):
        gstart(b, b)

    def body(g, carry):
        j0 = g * NBUF
        for b in range(NBUF):
            gwait(b)
            sstart(j0 + b, b)
        for b in range(NBUF):
            swait(b)
            gstart(j0 + NBUF + b, b)
        return carry

    lax.fori_loop(0, NGRP - 1, body, 0)

    j0 = (NGRP - 1) * NBUF
    for b in range(NBUF):
        gwait(b)
        sstart(j0 + b, b)
    for b in range(NBUF):
        swait(b)


def kernel(word_ids, table):
    ids_r = word_ids.reshape(NW, SENT_W, L)
    mesh = plsc.VectorSubcoreMesh(core_axis_name="c", subcore_axis_name="s")
    emb = functools.partial(
        pl.kernel,
        mesh=mesh,
        out_type=jax.ShapeDtypeStruct((B, L, DIM), jnp.float32),
        scratch_types=[
            pltpu.VMEM((SENT_W, L), jnp.int32),
            pltpu.VMEM((NBUF, CS, L, DIM), jnp.float32),
        ] + [pltpu.SemaphoreType.DMA] * (2 * NBUF),
    )(_emb_body)
    return emb(ids_r, table)


# speedup vs baseline: 7.7355x; 1.0023x over previous
import functools

import jax
import jax.numpy as jnp
from jax import lax
from jax.experimental import pallas as pl
from jax.experimental.pallas import tpu as pltpu
from jax.experimental.pallas import tpu_sc as plsc

B = 4096
L = 50
DIM = 128
NC = 2
NS = 16
NW = NC * NS
SENT_W = B // NW
CS = 2
NCHUNK = SENT_W // CS
NBUF = 4
NGRP = NCHUNK // NBUF


def _emb_body(ids_hbm, table_hbm, out_hbm, idx_v, rows_v, *sems):
    gsems = sems[:NBUF]
    ssems = sems[NBUF:]
    wid = lax.axis_index("s") * NC + lax.axis_index("c")
    base = wid * SENT_W
    pltpu.sync_copy(ids_hbm.at[wid], idx_v)

    def gstart(j, b):
        for s in range(CS):
            pltpu.make_async_copy(
                table_hbm.at[idx_v.at[j * CS + s]], rows_v.at[b, s],
                gsems[b]).start()

    def gwait(b):
        for s in range(CS):
            pltpu.make_async_copy(
                table_hbm.at[idx_v.at[0]], rows_v.at[b, s], gsems[b]).wait()

    def sstart(j, b):
        pltpu.make_async_copy(
            rows_v.at[b], out_hbm.at[pl.ds(base + j * CS, CS)],
            ssems[b]).start()

    def swait(b):
        pltpu.make_async_copy(
            rows_v.at[b], out_hbm.at[pl.ds(base, CS)], ssems[b]).wait()

    for b in range(NBUF):
        gstart(b, b)

    def body(g, carry):
        j0 = g * NBUF
        for b in range(NBUF):
            gwait(b)
            sstart(j0 + b, b)
        for b in range(NBUF):
            swait(b)
            gstart(j0 + NBUF + b, b)
        return carry

    lax.fori_loop(0, NGRP - 1, body, 0)

    j0 = (NGRP - 1) * NBUF
    for b in range(NBUF):
        gwait(b)
        sstart(j0 + b, b)
    for b in range(NBUF):
        swait(b)


def kernel(word_ids, table):
    ids_r = word_ids.reshape(NW, SENT_W, L)
    mesh = plsc.VectorSubcoreMesh(core_axis_name="c", subcore_axis_name="s")
    emb = functools.partial(
        pl.kernel,
        mesh=mesh,
        out_type=jax.ShapeDtypeStruct((B, L, DIM), jnp.float32),
        compiler_params=pltpu.CompilerParams(use_tc_tiling_on_sc=True),
        scratch_types=[
            pltpu.VMEM((SENT_W, L), jnp.int32),
            pltpu.VMEM((NBUF, CS, L, DIM), jnp.float32),
        ] + [pltpu.SemaphoreType.DMA] * (2 * NBUF),
    )(_emb_body)
    return emb(ids_r, table)
